# Initial kernel scaffold; baseline (speedup 1.0000x reference)
import jax
import jax.numpy as jnp
from jax.experimental import pallas as pl


def kernel(emds, target_rel, gru_bias, W_ihf, W_hhf, b_ihf, b_hhf, W_ihb, W_hhb, b_ihb, b_hhb, W3, b3, W1, b1, W2, b2, head_idx, tail_idx, num_graphs):
    raise NotImplementedError("write your pallas kernel here")


# speedup vs baseline: 10.6483x; 10.6483x over previous
import functools

import jax
import jax.numpy as jnp
from jax.experimental import pallas as pl
from jax.experimental.pallas import tpu as pltpu


def _h0_kernel(x_ref, h0_ref):
    i = pl.program_id(0)
    blk_max = jnp.max(x_ref[...], axis=0)

    @pl.when(i == 0)
    def _():
        h0_ref[...] = blk_max

    @pl.when(i > 0)
    def _():
        h0_ref[...] = jnp.maximum(h0_ref[...], blk_max)


def _gru_dir_step(x, h, wih_ref, whh_ref, bih_ref, bhh_ref, hdim):
    gi = jnp.dot(x, wih_ref[...], preferred_element_type=jnp.float32) + bih_ref[...]
    gh = jnp.dot(h, whh_ref[...], preferred_element_type=jnp.float32) + bhh_ref[...]
    r = jax.nn.sigmoid(gi[:, :hdim] + gh[:, :hdim])
    z = jax.nn.sigmoid(gi[:, hdim:2 * hdim] + gh[:, hdim:2 * hdim])
    n = jnp.tanh(gi[:, 2 * hdim:] + r * gh[:, 2 * hdim:])
    return (1.0 - z) * n + z * h


def _scan_kernel(xf_ref, xb_ref, h0_ref, gbias_ref,
                 wihf_ref, whhf_ref, bihf_ref, bhhf_ref,
                 wihb_ref, whhb_ref, bihb_ref, bhhb_ref,
                 outf_ref, outb_ref, hf_s, hb_s, *, steps, hdim):
    t = pl.program_id(0)

    @pl.when(t == 0)
    def _():
        hf_s[...] = h0_ref[...]
        hb_s[...] = h0_ref[...]

    for i in range(steps):
        xf = jnp.maximum(xf_ref[i] + gbias_ref[...], 0.0)
        hf = _gru_dir_step(xf, hf_s[...], wihf_ref, whhf_ref, bihf_ref,
                           bhhf_ref, hdim)
        hf_s[...] = hf
        outf_ref[i] = hf

        j = steps - 1 - i
        xb = jnp.maximum(xb_ref[j] + gbias_ref[...], 0.0)
        hb = _gru_dir_step(xb, hb_s[...], wihb_ref, whhb_ref, bihb_ref,
                           bhhb_ref, hdim)
        hb_s[...] = hb
        outb_ref[j] = hb


def _epilogue_kernel(lh_ref, bh_ref, lt_ref, bt_ref,
                     outfh_ref, outbh_ref, outft_ref, outbt_ref,
                     rel_ref, w3a_ref, w3b_ref, b3_ref, w1t_ref, b1_ref,
                     w2t_ref, s_ref, hf_s, hb_s, tf_s, tb_s, *, nb):
    g = pl.program_id(0)

    @pl.when(g == 0)
    def _():
        hf_s[...] = jnp.zeros_like(hf_s)
        hb_s[...] = jnp.zeros_like(hb_s)
        tf_s[...] = jnp.zeros_like(tf_s)
        tb_s[...] = jnp.zeros_like(tb_s)

    bh = bh_ref[g]
    bt = bt_ref[g]
    hf_s[pl.ds(g, 1), :] = outfh_ref[0, pl.ds(bh, 1), :]
    hb_s[pl.ds(g, 1), :] = outbh_ref[0, pl.ds(bh, 1), :]
    tf_s[pl.ds(g, 1), :] = outft_ref[0, pl.ds(bt, 1), :]
    tb_s[pl.ds(g, 1), :] = outbt_ref[0, pl.ds(bt, 1), :]

    @pl.when(g == nb - 1)
    def _():
        head = jnp.maximum(
            jnp.dot(hf_s[...], w3a_ref[...], preferred_element_type=jnp.float32)
            + jnp.dot(hb_s[...], w3b_ref[...], preferred_element_type=jnp.float32)
            + b3_ref[...], 0.0)
        tail = jnp.maximum(
            jnp.dot(tf_s[...], w3a_ref[...], preferred_element_type=jnp.float32)
            + jnp.dot(tb_s[...], w3b_ref[...], preferred_element_type=jnp.float32)
            + b3_ref[...], 0.0)
        feat = head + rel_ref[...] - tail
        s1 = jnp.dot(feat, w1t_ref[...], preferred_element_type=jnp.float32) + b1_ref[...]
        s_ref[...] = jnp.dot(s1, w2t_ref[...], preferred_element_type=jnp.float32)


def kernel(emds, target_rel, gru_bias, W_ihf, W_hhf, b_ihf, b_hhf, W_ihb,
           W_hhb, b_ihb, b_hhb, W3, b3, W1, b1, W2, b2, head_idx, tail_idx,
           num_graphs):
    n, h = emds.shape
    b = target_rel.shape[0]
    L = n // b
    bp = ((b + 7) // 8) * 8

    xt = jnp.pad(emds.reshape(b, L, h), ((0, bp - b), (0, 0), (0, 0)))
    xt = jnp.transpose(xt, (1, 0, 2))

    f32 = jnp.float32

    CA = 20
    while L % CA:
        CA -= 1
    h0 = pl.pallas_call(
        _h0_kernel,
        grid=(L // CA,),
        in_specs=[pl.BlockSpec((CA, bp, h), lambda i: (i, 0, 0))],
        out_specs=pl.BlockSpec((bp, h), lambda i: (0, 0)),
        out_shape=jax.ShapeDtypeStruct((bp, h), f32),
    )(xt)

    C = 5
    while L % C:
        C -= 1
    nblocks = L // C
    wihf = W_ihf.T
    whhf = W_hhf.T
    wihb = W_ihb.T
    whhb = W_hhb.T
    row = lambda v: v.reshape(1, -1)
    scan = pl.pallas_call(
        functools.partial(_scan_kernel, steps=C, hdim=h),
        grid=(nblocks,),
        in_specs=[
            pl.BlockSpec((C, bp, h), lambda t: (t, 0, 0)),
            pl.BlockSpec((C, bp, h), lambda t, nb=nblocks: (nb - 1 - t, 0, 0)),
            pl.BlockSpec((bp, h), lambda t: (0, 0)),
            pl.BlockSpec((1, h), lambda t: (0, 0)),
            pl.BlockSpec((h, 3 * h), lambda t: (0, 0)),
            pl.BlockSpec((h, 3 * h), lambda t: (0, 0)),
            pl.BlockSpec((1, 3 * h), lambda t: (0, 0)),
            pl.BlockSpec((1, 3 * h), lambda t: (0, 0)),
            pl.BlockSpec((h, 3 * h), lambda t: (0, 0)),
            pl.BlockSpec((h, 3 * h), lambda t: (0, 0)),
            pl.BlockSpec((1, 3 * h), lambda t: (0, 0)),
            pl.BlockSpec((1, 3 * h), lambda t: (0, 0)),
        ],
        out_specs=[
            pl.BlockSpec((C, bp, h), lambda t: (t, 0, 0)),
            pl.BlockSpec((C, bp, h), lambda t, nb=nblocks: (nb - 1 - t, 0, 0)),
        ],
        out_shape=[
            jax.ShapeDtypeStruct((L, bp, h), f32),
            jax.ShapeDtypeStruct((L, bp, h), f32),
        ],
        scratch_shapes=[pltpu.VMEM((bp, h), f32), pltpu.VMEM((bp, h), f32)],
    )(xt, xt, h0, row(gru_bias),
      wihf, whhf, row(b_ihf), row(b_hhf),
      wihb, whhb, row(b_ihb), row(b_hhb))
    out_f, out_b = scan

    lh = (head_idx % L).astype(jnp.int32)
    bh = (head_idx // L).astype(jnp.int32)
    lt = (tail_idx % L).astype(jnp.int32)
    bt = (tail_idx // L).astype(jnp.int32)
    rel_pad = jnp.pad(target_rel, ((0, bp - b), (0, 0)))
    w3t = W3.T
    w2t_pad = jnp.zeros((h, h), f32).at[:, 0].set(W2[0])

    grid_spec = pltpu.PrefetchScalarGridSpec(
        num_scalar_prefetch=4,
        grid=(b,),
        in_specs=[
            pl.BlockSpec((1, bp, h), lambda g, lh, bh, lt, bt: (lh[g], 0, 0)),
            pl.BlockSpec((1, bp, h), lambda g, lh, bh, lt, bt: (lh[g], 0, 0)),
            pl.BlockSpec((1, bp, h), lambda g, lh, bh, lt, bt: (lt[g], 0, 0)),
            pl.BlockSpec((1, bp, h), lambda g, lh, bh, lt, bt: (lt[g], 0, 0)),
            pl.BlockSpec((bp, h), lambda g, *_: (0, 0)),
            pl.BlockSpec((h, h), lambda g, *_: (0, 0)),
            pl.BlockSpec((h, h), lambda g, *_: (0, 0)),
            pl.BlockSpec((1, h), lambda g, *_: (0, 0)),
            pl.BlockSpec((h, h), lambda g, *_: (0, 0)),
            pl.BlockSpec((1, h), lambda g, *_: (0, 0)),
            pl.BlockSpec((h, h), lambda g, *_: (0, 0)),
        ],
        out_specs=pl.BlockSpec((bp, h), lambda g, *_: (0, 0)),
        scratch_shapes=[pltpu.VMEM((bp, h), f32)] * 4,
    )
    s2 = pl.pallas_call(
        functools.partial(_epilogue_kernel, nb=b),
        grid_spec=grid_spec,
        out_shape=jax.ShapeDtypeStruct((bp, h), f32),
    )(lh, bh, lt, bt, out_f, out_b, out_f, out_b,
      rel_pad, w3t[:h], w3t[h:], row(b3), W1.T, row(b1), w2t_pad)

    scores = s2[:b, :1] + b2
    return scores


# speedup vs baseline: 12.7206x; 1.1946x over previous
import functools

import jax
import jax.numpy as jnp
from jax import lax
from jax.experimental import pallas as pl
from jax.experimental.pallas import tpu as pltpu
from jax.experimental.pallas import tpu_sc as plsc

_NC, _NS = 2, 16
_NW = _NC * _NS


def _h0_kernel(x_ref, h0_ref):
    i = pl.program_id(0)
    blk_max = jnp.max(x_ref[...], axis=0)

    @pl.when(i == 0)
    def _():
        h0_ref[...] = blk_max

    @pl.when(i > 0)
    def _():
        h0_ref[...] = jnp.maximum(h0_ref[...], blk_max)


def _gru_dir_step(x, h, wih_ref, whh_ref, bih_ref, bhh_ref, hdim):
    gi = jnp.dot(x, wih_ref[...], preferred_element_type=jnp.float32) + bih_ref[...]
    gh = jnp.dot(h, whh_ref[...], preferred_element_type=jnp.float32) + bhh_ref[...]
    r = jax.nn.sigmoid(gi[:, :hdim] + gh[:, :hdim])
    z = jax.nn.sigmoid(gi[:, hdim:2 * hdim] + gh[:, hdim:2 * hdim])
    n = jnp.tanh(gi[:, 2 * hdim:] + r * gh[:, 2 * hdim:])
    return (1.0 - z) * n + z * h


def _scan_kernel(xf_ref, xb_ref, h0_ref, gbias_ref,
                 wihf_ref, whhf_ref, bihf_ref, bhhf_ref,
                 wihb_ref, whhb_ref, bihb_ref, bhhb_ref,
                 outf_ref, outb_ref, hf_s, hb_s, *, steps, hdim):
    t = pl.program_id(0)

    @pl.when(t == 0)
    def _():
        hf_s[...] = h0_ref[...]
        hb_s[...] = h0_ref[...]

    for i in range(steps):
        xf = jnp.maximum(xf_ref[i] + gbias_ref[...], 0.0)
        hf = _gru_dir_step(xf, hf_s[...], wihf_ref, whhf_ref, bihf_ref,
                           bhhf_ref, hdim)
        hf_s[...] = hf
        outf_ref[i] = hf

        j = steps - 1 - i
        xb = jnp.maximum(xb_ref[j] + gbias_ref[...], 0.0)
        hb = _gru_dir_step(xb, hb_s[...], wihb_ref, whhb_ref, bihb_ref,
                           bhhb_ref, hdim)
        hb_s[...] = hb
        outb_ref[j] = hb


def _score_kernel(gf_ref, gb_ref, rel_ref, w3a_ref, w3b_ref, b3_ref,
                  w1t_ref, b1_ref, w2t_ref, s_ref, *, rows):
    hf = gf_ref[:rows]
    tf = gf_ref[rows:]
    hb = gb_ref[:rows]
    tb = gb_ref[rows:]
    head = jnp.maximum(
        jnp.dot(hf, w3a_ref[...], preferred_element_type=jnp.float32)
        + jnp.dot(hb, w3b_ref[...], preferred_element_type=jnp.float32)
        + b3_ref[...], 0.0)
    tail = jnp.maximum(
        jnp.dot(tf, w3a_ref[...], preferred_element_type=jnp.float32)
        + jnp.dot(tb, w3b_ref[...], preferred_element_type=jnp.float32)
        + b3_ref[...], 0.0)
    feat = head + rel_ref[...] - tail
    s1 = jnp.dot(feat, w1t_ref[...], preferred_element_type=jnp.float32) + b1_ref[...]
    s_ref[...] = jnp.dot(s1, w2t_ref[...], preferred_element_type=jnp.float32)


def _make_sc_gather(rows_total, h, rpw):
    mesh = plsc.VectorSubcoreMesh(core_axis_name="c", subcore_axis_name="s",
                                  num_cores=_NC, num_subcores=_NS)

    @functools.partial(
        pl.kernel, mesh=mesh,
        out_type=[jax.ShapeDtypeStruct((rows_total, h), jnp.float32),
                  jax.ShapeDtypeStruct((rows_total, h), jnp.float32)],
        scratch_types=[pltpu.VMEM((rpw,), jnp.int32),
                       pltpu.VMEM((rpw, h), jnp.float32),
                       pltpu.VMEM((rpw, h), jnp.float32),
                       pltpu.SemaphoreType.DMA,
                       pltpu.SemaphoreType.DMA],
    )
    def gather_k(outf_hbm, outb_hbm, idx_hbm, gf_hbm, gb_hbm,
                 idx_v, rf_v, rb_v, semf, semb):
        wid = lax.axis_index("s") * _NC + lax.axis_index("c")
        base = wid * rpw
        pltpu.sync_copy(idx_hbm.at[pl.ds(base, rpw)], idx_v)
        cf = pltpu.async_copy(outf_hbm.at[idx_v], rf_v, semf)
        cb = pltpu.async_copy(outb_hbm.at[idx_v], rb_v, semb)
        cf.wait()
        cb.wait()
        pltpu.sync_copy(rf_v, gf_hbm.at[pl.ds(base, rpw)])
        pltpu.sync_copy(rb_v, gb_hbm.at[pl.ds(base, rpw)])

    return gather_k


def kernel(emds, target_rel, gru_bias, W_ihf, W_hhf, b_ihf, b_hhf, W_ihb,
           W_hhb, b_ihb, b_hhb, W3, b3, W1, b1, W2, b2, head_idx, tail_idx,
           num_graphs):
    n, h = emds.shape
    b = target_rel.shape[0]
    L = n // b
    bp = ((b + 7) // 8) * 8

    xt = jnp.pad(emds.reshape(b, L, h), ((0, bp - b), (0, 0), (0, 0)))
    xt = jnp.transpose(xt, (1, 0, 2))

    f32 = jnp.float32

    CA = 20
    while L % CA:
        CA -= 1
    h0 = pl.pallas_call(
        _h0_kernel,
        grid=(L // CA,),
        in_specs=[pl.BlockSpec((CA, bp, h), lambda i: (i, 0, 0))],
        out_specs=pl.BlockSpec((bp, h), lambda i: (0, 0)),
        out_shape=jax.ShapeDtypeStruct((bp, h), f32),
    )(xt)

    C = 5
    while L % C:
        C -= 1
    nblocks = L // C
    row = lambda v: v.reshape(1, -1)
    scan = pl.pallas_call(
        functools.partial(_scan_kernel, steps=C, hdim=h),
        grid=(nblocks,),
        in_specs=[
            pl.BlockSpec((C, bp, h), lambda t: (t, 0, 0)),
            pl.BlockSpec((C, bp, h), lambda t, nb=nblocks: (nb - 1 - t, 0, 0)),
            pl.BlockSpec((bp, h), lambda t: (0, 0)),
            pl.BlockSpec((1, h), lambda t: (0, 0)),
            pl.BlockSpec((h, 3 * h), lambda t: (0, 0)),
            pl.BlockSpec((h, 3 * h), lambda t: (0, 0)),
            pl.BlockSpec((1, 3 * h), lambda t: (0, 0)),
            pl.BlockSpec((1, 3 * h), lambda t: (0, 0)),
            pl.BlockSpec((h, 3 * h), lambda t: (0, 0)),
            pl.BlockSpec((h, 3 * h), lambda t: (0, 0)),
            pl.BlockSpec((1, 3 * h), lambda t: (0, 0)),
            pl.BlockSpec((1, 3 * h), lambda t: (0, 0)),
        ],
        out_specs=[
            pl.BlockSpec((C, bp, h), lambda t: (t, 0, 0)),
            pl.BlockSpec((C, bp, h), lambda t, nb=nblocks: (nb - 1 - t, 0, 0)),
        ],
        out_shape=[
            jax.ShapeDtypeStruct((L, bp, h), f32),
            jax.ShapeDtypeStruct((L, bp, h), f32),
        ],
        scratch_shapes=[pltpu.VMEM((bp, h), f32), pltpu.VMEM((bp, h), f32)],
    )(xt, xt, h0, row(gru_bias),
      W_ihf.T, W_hhf.T, row(b_ihf), row(b_hhf),
      W_ihb.T, W_hhb.T, row(b_ihb), row(b_hhb))
    out_f, out_b = scan

    sec = 128
    rows_total = 2 * sec
    rpw = rows_total // _NW
    fidx = (head_idx % L).astype(jnp.int32) * bp + (head_idx // L).astype(jnp.int32)
    tidx = (tail_idx % L).astype(jnp.int32) * bp + (tail_idx // L).astype(jnp.int32)
    idx = jnp.zeros((rows_total,), jnp.int32)
    idx = idx.at[:b].set(fidx).at[sec:sec + b].set(tidx)

    gf, gb = _make_sc_gather(rows_total, h, rpw)(
        out_f.reshape(L * bp, h), out_b.reshape(L * bp, h), idx)

    rel_pad = jnp.pad(target_rel, ((0, sec - b), (0, 0)))
    w3t = W3.T
    w2t_pad = jnp.zeros((h, h), f32).at[:, 0].set(W2[0])
    s2 = pl.pallas_call(
        functools.partial(_score_kernel, rows=sec),
        in_specs=[pl.BlockSpec((rows_total, h), lambda: (0, 0)),
                  pl.BlockSpec((rows_total, h), lambda: (0, 0)),
                  pl.BlockSpec((sec, h), lambda: (0, 0)),
                  pl.BlockSpec((h, h), lambda: (0, 0)),
                  pl.BlockSpec((h, h), lambda: (0, 0)),
                  pl.BlockSpec((1, h), lambda: (0, 0)),
                  pl.BlockSpec((h, h), lambda: (0, 0)),
                  pl.BlockSpec((1, h), lambda: (0, 0)),
                  pl.BlockSpec((h, h), lambda: (0, 0))],
        out_specs=pl.BlockSpec((sec, h), lambda: (0, 0)),
        out_shape=jax.ShapeDtypeStruct((sec, h), f32),
    )(gf, gb, rel_pad, w3t[:h], w3t[h:], row(b3), W1.T, row(b1), w2t_pad)

    scores = s2[:b, :1] + b2
    return scores


# speedup vs baseline: 14.7195x; 1.1571x over previous
import functools

import jax
import jax.numpy as jnp
from jax import lax
from jax.experimental import pallas as pl
from jax.experimental.pallas import tpu as pltpu
from jax.experimental.pallas import tpu_sc as plsc

_NC, _NS = 2, 16
_NW = _NC * _NS


def _prep_kernel(x_ref, xt_ref, h0_ref):
    x = x_ref[...]
    h0_ref[...] = jnp.max(x, axis=1)
    xt_ref[...] = jnp.swapaxes(x, 0, 1)


def _gru_dir_step(x, h, wih_ref, whh_ref, bih_ref, bhh_ref, hdim):
    gi = jnp.dot(x, wih_ref[...], preferred_element_type=jnp.float32) + bih_ref[...]
    gh = jnp.dot(h, whh_ref[...], preferred_element_type=jnp.float32) + bhh_ref[...]
    r = jax.nn.sigmoid(gi[:, :hdim] + gh[:, :hdim])
    z = jax.nn.sigmoid(gi[:, hdim:2 * hdim] + gh[:, hdim:2 * hdim])
    n = jnp.tanh(gi[:, 2 * hdim:] + r * gh[:, 2 * hdim:])
    return (1.0 - z) * n + z * h


def _scan_kernel(xf_ref, xb_ref, h0_ref, gbias_ref,
                 wihf_ref, whhf_ref, bihf_ref, bhhf_ref,
                 wihb_ref, whhb_ref, bihb_ref, bhhb_ref,
                 outf_ref, outb_ref, hf_s, hb_s, *, steps, hdim):
    t = pl.program_id(0)

    @pl.when(t == 0)
    def _():
        hf_s[...] = h0_ref[...]
        hb_s[...] = h0_ref[...]

    for i in range(steps):
        xf = jnp.maximum(xf_ref[i] + gbias_ref[...], 0.0)
        hf = _gru_dir_step(xf, hf_s[...], wihf_ref, whhf_ref, bihf_ref,
                           bhhf_ref, hdim)
        hf_s[...] = hf
        outf_ref[i] = hf

        j = steps - 1 - i
        xb = jnp.maximum(xb_ref[j] + gbias_ref[...], 0.0)
        hb = _gru_dir_step(xb, hb_s[...], wihb_ref, whhb_ref, bihb_ref,
                           bhhb_ref, hdim)
        hb_s[...] = hb
        outb_ref[j] = hb


def _score_kernel(gf_ref, gb_ref, rel_ref, w3a_ref, w3b_ref, b3_ref,
                  w1t_ref, b1_ref, w2t_ref, s_ref, *, rows):
    hf = gf_ref[:rows]
    tf = gf_ref[rows:]
    hb = gb_ref[:rows]
    tb = gb_ref[rows:]
    head = jnp.maximum(
        jnp.dot(hf, w3a_ref[...], preferred_element_type=jnp.float32)
        + jnp.dot(hb, w3b_ref[...], preferred_element_type=jnp.float32)
        + b3_ref[...], 0.0)
    tail = jnp.maximum(
        jnp.dot(tf, w3a_ref[...], preferred_element_type=jnp.float32)
        + jnp.dot(tb, w3b_ref[...], preferred_element_type=jnp.float32)
        + b3_ref[...], 0.0)
    feat = head + rel_ref[...] - tail
    s1 = jnp.dot(feat, w1t_ref[...], preferred_element_type=jnp.float32) + b1_ref[...]
    s_ref[...] = jnp.dot(s1, w2t_ref[...], preferred_element_type=jnp.float32)


def _make_sc_gather(rows_total, h, rpw):
    mesh = plsc.VectorSubcoreMesh(core_axis_name="c", subcore_axis_name="s",
                                  num_cores=_NC, num_subcores=_NS)

    @functools.partial(
        pl.kernel, mesh=mesh,
        out_type=[jax.ShapeDtypeStruct((rows_total, h), jnp.float32),
                  jax.ShapeDtypeStruct((rows_total, h), jnp.float32)],
        scratch_types=[pltpu.VMEM((rpw,), jnp.int32),
                       pltpu.VMEM((rpw, h), jnp.float32),
                       pltpu.VMEM((rpw, h), jnp.float32),
                       pltpu.SemaphoreType.DMA,
                       pltpu.SemaphoreType.DMA],
    )
    def gather_k(outf_hbm, outb_hbm, idx_hbm, gf_hbm, gb_hbm,
                 idx_v, rf_v, rb_v, semf, semb):
        wid = lax.axis_index("s") * _NC + lax.axis_index("c")
        base = wid * rpw
        pltpu.sync_copy(idx_hbm.at[pl.ds(base, rpw)], idx_v)
        cf = pltpu.async_copy(outf_hbm.at[idx_v], rf_v, semf)
        cb = pltpu.async_copy(outb_hbm.at[idx_v], rb_v, semb)
        cf.wait()
        cb.wait()
        pltpu.sync_copy(rf_v, gf_hbm.at[pl.ds(base, rpw)])
        pltpu.sync_copy(rb_v, gb_hbm.at[pl.ds(base, rpw)])

    return gather_k


def kernel(emds, target_rel, gru_bias, W_ihf, W_hhf, b_ihf, b_hhf, W_ihb,
           W_hhb, b_ihb, b_hhb, W3, b3, W1, b1, W2, b2, head_idx, tail_idx,
           num_graphs):
    n, h = emds.shape
    b = target_rel.shape[0]
    L = n // b
    bp = ((b + 7) // 8) * 8

    f32 = jnp.float32

    xt, h0 = pl.pallas_call(
        _prep_kernel,
        grid=(bp // 8,),
        in_specs=[pl.BlockSpec((8, L, h), lambda i: (i, 0, 0))],
        out_specs=[pl.BlockSpec((L, 8, h), lambda i: (0, i, 0)),
                   pl.BlockSpec((8, h), lambda i: (i, 0))],
        out_shape=[jax.ShapeDtypeStruct((L, bp, h), f32),
                   jax.ShapeDtypeStruct((bp, h), f32)],
    )(emds.reshape(b, L, h))

    C = 5
    while L % C:
        C -= 1
    nblocks = L // C
    row = lambda v: v.reshape(1, -1)
    scan = pl.pallas_call(
        functools.partial(_scan_kernel, steps=C, hdim=h),
        grid=(nblocks,),
        in_specs=[
            pl.BlockSpec((C, bp, h), lambda t: (t, 0, 0)),
            pl.BlockSpec((C, bp, h), lambda t, nb=nblocks: (nb - 1 - t, 0, 0)),
            pl.BlockSpec((bp, h), lambda t: (0, 0)),
            pl.BlockSpec((1, h), lambda t: (0, 0)),
            pl.BlockSpec((h, 3 * h), lambda t: (0, 0)),
            pl.BlockSpec((h, 3 * h), lambda t: (0, 0)),
            pl.BlockSpec((1, 3 * h), lambda t: (0, 0)),
            pl.BlockSpec((1, 3 * h), lambda t: (0, 0)),
            pl.BlockSpec((h, 3 * h), lambda t: (0, 0)),
            pl.BlockSpec((h, 3 * h), lambda t: (0, 0)),
            pl.BlockSpec((1, 3 * h), lambda t: (0, 0)),
            pl.BlockSpec((1, 3 * h), lambda t: (0, 0)),
        ],
        out_specs=[
            pl.BlockSpec((C, bp, h), lambda t: (t, 0, 0)),
            pl.BlockSpec((C, bp, h), lambda t, nb=nblocks: (nb - 1 - t, 0, 0)),
        ],
        out_shape=[
            jax.ShapeDtypeStruct((L, bp, h), f32),
            jax.ShapeDtypeStruct((L, bp, h), f32),
        ],
        scratch_shapes=[pltpu.VMEM((bp, h), f32), pltpu.VMEM((bp, h), f32)],
    )(xt, xt, h0, row(gru_bias),
      W_ihf.T, W_hhf.T, row(b_ihf), row(b_hhf),
      W_ihb.T, W_hhb.T, row(b_ihb), row(b_hhb))
    out_f, out_b = scan

    sec = 128
    rows_total = 2 * sec
    rpw = rows_total // _NW
    fidx = (head_idx % L).astype(jnp.int32) * bp + (head_idx // L).astype(jnp.int32)
    tidx = (tail_idx % L).astype(jnp.int32) * bp + (tail_idx // L).astype(jnp.int32)
    idx = jnp.zeros((rows_total,), jnp.int32)
    idx = idx.at[:b].set(fidx).at[sec:sec + b].set(tidx)

    gf, gb = _make_sc_gather(rows_total, h, rpw)(
        out_f.reshape(L * bp, h), out_b.reshape(L * bp, h), idx)

    rel_pad = jnp.pad(target_rel, ((0, sec - b), (0, 0)))
    w3t = W3.T
    w2t_pad = jnp.zeros((h, h), f32).at[:, 0].set(W2[0])
    s2 = pl.pallas_call(
        functools.partial(_score_kernel, rows=sec),
        in_specs=[pl.BlockSpec((rows_total, h), lambda: (0, 0)),
                  pl.BlockSpec((rows_total, h), lambda: (0, 0)),
                  pl.BlockSpec((sec, h), lambda: (0, 0)),
                  pl.BlockSpec((h, h), lambda: (0, 0)),
                  pl.BlockSpec((h, h), lambda: (0, 0)),
                  pl.BlockSpec((1, h), lambda: (0, 0)),
                  pl.BlockSpec((h, h), lambda: (0, 0)),
                  pl.BlockSpec((1, h), lambda: (0, 0)),
                  pl.BlockSpec((h, h), lambda: (0, 0))],
        out_specs=pl.BlockSpec((sec, h), lambda: (0, 0)),
        out_shape=jax.ShapeDtypeStruct((sec, h), f32),
    )(gf, gb, rel_pad, w3t[:h], w3t[h:], row(b3), W1.T, row(b1), w2t_pad)

    scores = s2[:b, :1] + b2
    return scores
